# baseline (speedup 1.0000x reference)
import functools

import jax
import jax.numpy as jnp
from jax import lax
from jax.experimental import pallas as pl
from jax.experimental.pallas import tpu as pltpu
from jax.experimental.pallas import tpu_sc as plsc

E_PAD = 128
NEG = -1e30
BLK = 512
NW = 32
CH = 16


def _gate_all(scores):
    bn = scores.shape[0]
    col = lax.broadcasted_iota(jnp.int32, (bn, E_PAD), 1)
    m1 = jnp.max(scores, axis=1, keepdims=True)
    idx1 = jnp.min(jnp.where(scores == m1, col, E_PAD), axis=1, keepdims=True)
    mask1 = col == idx1
    s2 = jnp.where(mask1, NEG, scores)
    m2 = jnp.max(s2, axis=1, keepdims=True)
    idx2 = jnp.min(jnp.where(s2 == m2, col, E_PAD), axis=1, keepdims=True)
    mask2 = col == idx2
    es = jnp.exp(scores - m1)
    z = jnp.sum(es, axis=1, keepdims=True)
    soft = es / z
    top = jnp.where(mask1 | mask2, soft, 0.0)
    topsum = jnp.sum(top, axis=1, keepdims=True)
    return top / (topsum + 1e-8), mask1, mask2


def _cumsum_shift(v, axis):
    n = v.shape[axis]
    k = 1
    while k < n:
        if axis == 0:
            shifted = jnp.concatenate(
                [jnp.zeros((k,) + v.shape[1:], v.dtype), v[:-k]], axis=0)
        else:
            shifted = jnp.concatenate(
                [jnp.zeros(v.shape[:1] + (k,), v.dtype), v[:, :-k]], axis=1)
        v = v + shifted
        k *= 2
    return v


def _gate_body(xb, wgb, bgb, p1b, p2b, cntb):
    n = xb.shape[0]
    scores = jnp.dot(xb[...], wgb[...],
                     preferred_element_type=jnp.float32) + bgb[...]
    _, mask1, mask2 = _gate_all(scores)
    cnt = jnp.where(mask1 | mask2, 1.0, 0.0)
    pcum = _cumsum_shift(cnt, axis=0) - cnt
    counts = jnp.sum(cnt, axis=0, keepdims=True)
    gs = _cumsum_shift(counts, axis=1) - counts
    posf = gs + pcum
    p1 = jnp.sum(jnp.where(mask1, posf, 0.0), axis=1, keepdims=True)
    p2 = jnp.sum(jnp.where(mask2, posf, 0.0), axis=1, keepdims=True)
    p1b[...] = jnp.broadcast_to(p1.astype(jnp.int32), (n, E_PAD))
    p2b[...] = jnp.broadcast_to(p2.astype(jnp.int32), (n, E_PAD))
    cntb[...] = counts


def _gating(x, wg_pad, bg_pad):
    n_tok, d_in = x.shape
    return pl.pallas_call(
        _gate_body,
        grid=(1,),
        in_specs=[
            pl.BlockSpec((n_tok, d_in), lambda i: (0, 0)),
            pl.BlockSpec((d_in, E_PAD), lambda i: (0, 0)),
            pl.BlockSpec((1, E_PAD), lambda i: (0, 0)),
        ],
        out_specs=[
            pl.BlockSpec((n_tok, E_PAD), lambda i: (0, 0)),
            pl.BlockSpec((n_tok, E_PAD), lambda i: (0, 0)),
            pl.BlockSpec((1, E_PAD), lambda i: (0, 0)),
        ],
        out_shape=[
            jax.ShapeDtypeStruct((n_tok, E_PAD), jnp.int32),
            jax.ShapeDtypeStruct((n_tok, E_PAD), jnp.int32),
            jax.ShapeDtypeStruct((1, E_PAD), jnp.float32),
        ],
    )(x, wg_pad, bg_pad)


def _dispatch(x, pos1, pos2):
    n_tok, d_in = x.shape
    n_rows = 2 * n_tok
    per_w = n_tok // NW
    n_ch, ch = pos1.shape[1], pos1.shape[2]
    mesh = plsc.VectorSubcoreMesh(core_axis_name="c", subcore_axis_name="s")

    @functools.partial(
        pl.kernel, mesh=mesh,
        out_type=jax.ShapeDtypeStruct((n_rows, d_in), jnp.float32),
        scratch_types=[
            pltpu.VMEM((n_ch, ch), jnp.int32),
            pltpu.VMEM((n_ch, ch), jnp.int32),
            pltpu.VMEM((ch, d_in), jnp.float32),
            pltpu.VMEM((ch, d_in), jnp.float32),
            pltpu.SemaphoreType.DMA,
            pltpu.SemaphoreType.DMA,
            pltpu.SemaphoreType.DMA,
            pltpu.SemaphoreType.DMA,
        ],
    )
    def k2(x_hbm, p1_hbm, p2_hbm, xs_hbm, i1, i2, bufa, bufb, la, lb, sa, sb):
        wid = lax.axis_index("s") * 2 + lax.axis_index("c")
        pltpu.sync_copy(p1_hbm.at[wid], i1)
        pltpu.sync_copy(p2_hbm.at[wid], i2)
        base = wid * per_w
        bufs = (bufa, bufb)
        lsem = (la, lb)
        ssem = (sa, sb)
        loads = [None] * n_ch
        scats = [None] * n_ch

        def issue_load(c):
            loads[c] = pltpu.async_copy(
                x_hbm.at[pl.ds(base + c * ch, ch)], bufs[c & 1], lsem[c & 1])

        issue_load(0)
        for c in range(n_ch):
            p = c & 1
            loads[c].wait()
            h1 = pltpu.async_copy(bufs[p], xs_hbm.at[i1.at[c]], ssem[p])
            h2 = pltpu.async_copy(bufs[p], xs_hbm.at[i2.at[c]], ssem[p])
            scats[c] = (h1, h2)
            if c + 1 < n_ch:
                if c >= 1:
                    scats[c - 1][0].wait()
                    scats[c - 1][1].wait()
                issue_load(c + 1)
        if n_ch >= 2:
            scats[n_ch - 2][0].wait()
            scats[n_ch - 2][1].wait()
        scats[n_ch - 1][0].wait()
        scats[n_ch - 1][1].wait()

    return k2(x, pos1, pos2)


def _mlp_body(n_experts, tile_r, tile_e, tile_f, gs8, xb, wgb, bgb, w1b, b1b,
              w2b, b2b, ob):
    t = pl.program_id(0)
    e = tile_e[t]

    @pl.when(e < n_experts)
    def _():
        r = tile_r[t]
        bn = xb.shape[0]
        x = xb[...]
        rows = r * BLK + lax.broadcasted_iota(jnp.int32, (bn, 1), 0)
        erow = jnp.zeros((bn, 1), jnp.int32)
        for j in range(1, n_experts):
            erow = erow + jnp.where(rows >= gs8[j], 1, 0)
        scores = jnp.dot(x, wgb[...],
                         preferred_element_type=jnp.float32) + bgb[...]
        wfull, _, _ = _gate_all(scores)
        col = lax.broadcasted_iota(jnp.int32, (bn, E_PAD), 1)
        wcol = jnp.sum(jnp.where(col == e, wfull, 0.0), axis=1, keepdims=True)
        coef = jnp.where(erow == e, wcol, 0.0)
        h = jnp.maximum(
            jnp.dot(x.astype(jnp.bfloat16), w1b[0],
                    preferred_element_type=jnp.float32) + b1b[0],
            0.0)
        y = jnp.dot(h.astype(jnp.bfloat16), w2b[0],
                    preferred_element_type=jnp.float32) + b2b[0]
        contrib = coef * y

        @pl.when(tile_f[t] == 1)
        def _():
            ob[...] = contrib

        @pl.when(tile_f[t] == 0)
        def _():
            ob[...] += contrib


def _grouped_mlp(xs, wg_pad, bg_pad, W1, b1r, W2, b2r,
                 tile_r, tile_e, tile_f, gs8, t_max):
    n_rows, d_in = xs.shape
    n_experts, _, d_hid = W1.shape
    d_out = W2.shape[2]

    grid_spec = pltpu.PrefetchScalarGridSpec(
        num_scalar_prefetch=4,
        grid=(t_max,),
        in_specs=[
            pl.BlockSpec((BLK, d_in), lambda t, sr, se, sf, sg: (sr[t], 0)),
            pl.BlockSpec((d_in, E_PAD), lambda t, sr, se, sf, sg: (0, 0)),
            pl.BlockSpec((1, E_PAD), lambda t, sr, se, sf, sg: (0, 0)),
            pl.BlockSpec((1, d_in, d_hid),
                         lambda t, sr, se, sf, sg: (jnp.minimum(se[t], 5), 0, 0)),
            pl.BlockSpec((1, 1, d_hid),
                         lambda t, sr, se, sf, sg: (jnp.minimum(se[t], 5), 0, 0)),
            pl.BlockSpec((1, d_hid, d_out),
                         lambda t, sr, se, sf, sg: (jnp.minimum(se[t], 5), 0, 0)),
            pl.BlockSpec((1, 1, d_out),
                         lambda t, sr, se, sf, sg: (jnp.minimum(se[t], 5), 0, 0)),
        ],
        out_specs=pl.BlockSpec((BLK, d_out), lambda t, sr, se, sf, sg: (sr[t], 0)),
    )
    return pl.pallas_call(
        functools.partial(_mlp_body, n_experts),
        grid_spec=grid_spec,
        out_shape=jax.ShapeDtypeStruct((n_rows, d_out), jnp.float32),
        compiler_params=pltpu.CompilerParams(
            dimension_semantics=("arbitrary",)),
    )(tile_r, tile_e, tile_f, gs8, xs, wg_pad, bg_pad, W1, b1r, W2, b2r)


def _combine(ysw, pos1, pos2, n_tok):
    n_rows, d_out = ysw.shape
    per_w = n_tok // NW
    n_ch = per_w // CH
    mesh = plsc.VectorSubcoreMesh(core_axis_name="c", subcore_axis_name="s")

    @functools.partial(
        pl.kernel, mesh=mesh,
        out_type=jax.ShapeDtypeStruct((n_tok, d_out), jnp.float32),
        scratch_types=[
            pltpu.VMEM((n_ch, CH), jnp.int32),
            pltpu.VMEM((n_ch, CH), jnp.int32),
            pltpu.VMEM((CH, d_out), jnp.float32),
            pltpu.VMEM((CH, d_out), jnp.float32),
            pltpu.VMEM((CH, d_out), jnp.float32),
            pltpu.VMEM((CH, d_out), jnp.float32),
            pltpu.SemaphoreType.DMA,
            pltpu.SemaphoreType.DMA,
            pltpu.SemaphoreType.DMA,
            pltpu.SemaphoreType.DMA,
        ],
    )
    def k4(ys_hbm, p1_hbm, p2_hbm, out_hbm, ia, ib, ba0, bb0, ba1, bb1,
           g0, g1, st0, st1):
        wid = lax.axis_index("s") * 2 + lax.axis_index("c")
        pltpu.sync_copy(p1_hbm.at[wid], ia)
        pltpu.sync_copy(p2_hbm.at[wid], ib)
        base = wid * per_w
        babufs = (ba0, ba1)
        bbbufs = (bb0, bb1)
        gsem = (g0, g1)
        stsem = (st0, st1)
        gaths = [None] * n_ch
        stores = [None] * n_ch

        def issue_gathers(c):
            p = c & 1
            h1 = pltpu.async_copy(ys_hbm.at[ia.at[c]], babufs[p], gsem[p])
            h2 = pltpu.async_copy(ys_hbm.at[ib.at[c]], bbbufs[p], gsem[p])
            gaths[c] = (h1, h2)

        issue_gathers(0)
        for c in range(n_ch):
            p = c & 1
            gaths[c][0].wait()
            gaths[c][1].wait()
            if c + 1 < n_ch:
                if c >= 1:
                    stores[c - 1].wait()
                issue_gathers(c + 1)
            ba = babufs[p]
            bb = bbbufs[p]

            def add_row(j, _):
                for k in range(d_out // 16):
                    sl = pl.ds(k * 16, 16)
                    ba[j, sl] = ba[j, sl] + bb[j, sl]
                return 0

            lax.fori_loop(0, CH, add_row, 0)
            stores[c] = pltpu.async_copy(
                ba, out_hbm.at[pl.ds(base + c * CH, CH)], stsem[p])
        if n_ch >= 2:
            stores[n_ch - 2].wait()
        stores[n_ch - 1].wait()

    return k4(ysw, pos1, pos2)


def kernel(x, Wg, bg, W1, b1, W2, b2):
    n_tok, d_in = x.shape
    n_experts, _, d_hid = W1.shape
    d_out = W2.shape[2]
    n_rows = 2 * n_tok
    r_blocks = n_rows // BLK
    t_max = r_blocks + n_experts - 1

    wg_pad = jnp.zeros((d_in, E_PAD), jnp.float32).at[:, :n_experts].set(Wg)
    bg_pad = jnp.full((E_PAD,), NEG, jnp.float32).at[:n_experts].set(bg)
    bg_pad = bg_pad.reshape(1, E_PAD)
    b1r = b1.reshape(n_experts, 1, d_hid)
    b2r = b2.reshape(n_experts, 1, d_out)

    p1f, p2f, counts_f = _gating(x, wg_pad, bg_pad)
    pos1 = p1f[:, 0].reshape(NW, -1)
    pos2 = p2f[:, 0].reshape(NW, -1)
    ch_d = 32 if (n_tok // NW) % 32 == 0 else CH
    pos1_d = pos1.reshape(NW, -1, ch_d)
    pos2_d = pos2.reshape(NW, -1, ch_d)
    pos1_c = pos1.reshape(NW, -1, CH)
    pos2_c = pos2.reshape(NW, -1, CH)

    counts = counts_f[0, :n_experts].astype(jnp.int32)
    ends = jnp.cumsum(counts)
    starts = ends - counts
    gs8 = jnp.concatenate(
        [starts, jnp.full((8 - n_experts,), n_rows, jnp.int32)])
    first_t = starts // BLK
    last_t = jnp.where(counts > 0, (ends - 1) // BLK, first_t - 1)
    nt = jnp.maximum(last_t - first_t + 1, 0)
    nt_ends = jnp.cumsum(nt)
    nt_starts = nt_ends - nt
    t_ids = jnp.arange(t_max, dtype=jnp.int32)
    e_t = jnp.sum(t_ids[:, None] >= nt_ends[None, :], axis=1).astype(jnp.int32)
    e_c = jnp.minimum(e_t, n_experts - 1)
    r_t = jnp.where(e_t < n_experts,
                    first_t[e_c] + (t_ids - nt_starts[e_c]),
                    r_blocks - 1).astype(jnp.int32)
    f_t = jnp.concatenate(
        [jnp.ones((1,), jnp.int32),
         (r_t[1:] != r_t[:-1]).astype(jnp.int32)])

    xs = _dispatch(x, pos1_d, pos2_d)
    ysw = _grouped_mlp(xs, wg_pad, bg_pad,
                       W1.astype(jnp.bfloat16), b1r,
                       W2.astype(jnp.bfloat16), b2r,
                       r_t, e_t, f_t, gs8, t_max)
    return _combine(ysw, pos1_c, pos2_c, n_tok)


# speedup vs baseline: 1.2447x; 1.2447x over previous
import functools

import jax
import jax.numpy as jnp
from jax import lax
from jax.experimental import pallas as pl
from jax.experimental.pallas import tpu as pltpu
from jax.experimental.pallas import tpu_sc as plsc

E_PAD = 128
NEG = -1e30
BLK = 512
NW = 32
CH = 16


def _gate_all(scores):
    bn = scores.shape[0]
    col = lax.broadcasted_iota(jnp.int32, (bn, E_PAD), 1)
    m1 = jnp.max(scores, axis=1, keepdims=True)
    idx1 = jnp.min(jnp.where(scores == m1, col, E_PAD), axis=1, keepdims=True)
    mask1 = col == idx1
    s2 = jnp.where(mask1, NEG, scores)
    m2 = jnp.max(s2, axis=1, keepdims=True)
    idx2 = jnp.min(jnp.where(s2 == m2, col, E_PAD), axis=1, keepdims=True)
    mask2 = col == idx2
    es = jnp.exp(scores - m1)
    z = jnp.sum(es, axis=1, keepdims=True)
    soft = es / z
    top = jnp.where(mask1 | mask2, soft, 0.0)
    topsum = jnp.sum(top, axis=1, keepdims=True)
    return top / (topsum + 1e-8), mask1, mask2


def _cumsum_shift(v, axis):
    n = v.shape[axis]
    k = 1
    while k < n:
        if axis == 0:
            shifted = jnp.concatenate(
                [jnp.zeros((k,) + v.shape[1:], v.dtype), v[:-k]], axis=0)
        else:
            shifted = jnp.concatenate(
                [jnp.zeros(v.shape[:1] + (k,), v.dtype), v[:, :-k]], axis=1)
        v = v + shifted
        k *= 2
    return v


def _gate_body(xb, wgb, bgb, p1b, p2b, cntb):
    n = xb.shape[0]
    scores = jnp.dot(xb[...], wgb[...],
                     preferred_element_type=jnp.float32) + bgb[...]
    _, mask1, mask2 = _gate_all(scores)
    cnt = jnp.where(mask1 | mask2, 1.0, 0.0)
    pcum = _cumsum_shift(cnt, axis=0) - cnt
    counts = jnp.sum(cnt, axis=0, keepdims=True)
    pcnt = jnp.ceil(counts * (1.0 / BLK)) * BLK
    gs = _cumsum_shift(pcnt, axis=1) - pcnt
    posf = gs + pcum
    p1 = jnp.sum(jnp.where(mask1, posf, 0.0), axis=1, keepdims=True)
    p2 = jnp.sum(jnp.where(mask2, posf, 0.0), axis=1, keepdims=True)
    p1b[...] = jnp.broadcast_to(p1.astype(jnp.int32), (n, E_PAD))
    p2b[...] = jnp.broadcast_to(p2.astype(jnp.int32), (n, E_PAD))
    cntb[...] = counts


def _gating(x, wg_pad, bg_pad):
    n_tok, d_in = x.shape
    return pl.pallas_call(
        _gate_body,
        grid=(1,),
        in_specs=[
            pl.BlockSpec((n_tok, d_in), lambda i: (0, 0)),
            pl.BlockSpec((d_in, E_PAD), lambda i: (0, 0)),
            pl.BlockSpec((1, E_PAD), lambda i: (0, 0)),
        ],
        out_specs=[
            pl.BlockSpec((n_tok, E_PAD), lambda i: (0, 0)),
            pl.BlockSpec((n_tok, E_PAD), lambda i: (0, 0)),
            pl.BlockSpec((1, E_PAD), lambda i: (0, 0)),
        ],
        out_shape=[
            jax.ShapeDtypeStruct((n_tok, E_PAD), jnp.int32),
            jax.ShapeDtypeStruct((n_tok, E_PAD), jnp.int32),
            jax.ShapeDtypeStruct((1, E_PAD), jnp.float32),
        ],
    )(x, wg_pad, bg_pad)


def _dispatch(x, pos1, pos2, n_rows_pad):
    n_tok, d_in = x.shape
    n_rows = n_rows_pad
    per_w = n_tok // NW
    n_ch, ch = pos1.shape[1], pos1.shape[2]
    mesh = plsc.VectorSubcoreMesh(core_axis_name="c", subcore_axis_name="s")

    @functools.partial(
        pl.kernel, mesh=mesh,
        out_type=jax.ShapeDtypeStruct((n_rows, d_in), jnp.float32),
        scratch_types=[
            pltpu.VMEM((n_ch, ch), jnp.int32),
            pltpu.VMEM((n_ch, ch), jnp.int32),
            pltpu.VMEM((ch, d_in), jnp.float32),
            pltpu.VMEM((ch, d_in), jnp.float32),
            pltpu.SemaphoreType.DMA,
            pltpu.SemaphoreType.DMA,
            pltpu.SemaphoreType.DMA,
            pltpu.SemaphoreType.DMA,
        ],
    )
    def k2(x_hbm, p1_hbm, p2_hbm, xs_hbm, i1, i2, bufa, bufb, la, lb, sa, sb):
        wid = lax.axis_index("s") * 2 + lax.axis_index("c")
        pltpu.sync_copy(p1_hbm.at[wid], i1)
        pltpu.sync_copy(p2_hbm.at[wid], i2)
        base = wid * per_w
        bufs = (bufa, bufb)
        lsem = (la, lb)
        ssem = (sa, sb)
        loads = [None] * n_ch
        scats = [None] * n_ch

        def issue_load(c):
            loads[c] = pltpu.async_copy(
                x_hbm.at[pl.ds(base + c * ch, ch)], bufs[c & 1], lsem[c & 1])

        issue_load(0)
        for c in range(n_ch):
            p = c & 1
            loads[c].wait()
            h1 = pltpu.async_copy(bufs[p], xs_hbm.at[i1.at[c]], ssem[p])
            h2 = pltpu.async_copy(bufs[p], xs_hbm.at[i2.at[c]], ssem[p])
            scats[c] = (h1, h2)
            if c + 1 < n_ch:
                if c >= 1:
                    scats[c - 1][0].wait()
                    scats[c - 1][1].wait()
                issue_load(c + 1)
        if n_ch >= 2:
            scats[n_ch - 2][0].wait()
            scats[n_ch - 2][1].wait()
        scats[n_ch - 1][0].wait()
        scats[n_ch - 1][1].wait()

    return k2(x, pos1, pos2)


def _mlp_body(n_experts, tile_r, tile_ec, tile_e, gs8, xb, wgb, bgb, w1b, b1b,
              w2b, b2b, ob):
    t = pl.program_id(0)
    e = tile_e[t]

    @pl.when(e < n_experts)
    def _():
        r = tile_r[t]
        bn = xb.shape[0]
        x = xb[...]
        rows = r * BLK + lax.broadcasted_iota(jnp.int32, (bn, 1), 0)
        erow = jnp.zeros((bn, 1), jnp.int32)
        for j in range(1, n_experts):
            erow = erow + jnp.where(rows >= gs8[j], 1, 0)
        scores = jnp.dot(x, wgb[...],
                         preferred_element_type=jnp.float32) + bgb[...]
        wfull, _, _ = _gate_all(scores)
        col = lax.broadcasted_iota(jnp.int32, (bn, E_PAD), 1)
        wcol = jnp.sum(jnp.where(col == e, wfull, 0.0), axis=1, keepdims=True)
        coef = jnp.where(erow == e, wcol, 0.0)
        h = jnp.maximum(
            jnp.dot(x, w1b[0], preferred_element_type=jnp.float32) + b1b[0],
            0.0)
        y = jnp.dot(h, w2b[0], preferred_element_type=jnp.float32) + b2b[0]
        ob[...] = coef * y


def _grouped_mlp(xs, wg_pad, bg_pad, W1, b1r, W2, b2r,
                 tile_r, tile_ec, tile_e, gs8, t_max):
    n_rows, d_in = xs.shape
    n_experts, _, d_hid = W1.shape
    d_out = W2.shape[2]

    grid_spec = pltpu.PrefetchScalarGridSpec(
        num_scalar_prefetch=4,
        grid=(t_max,),
        in_specs=[
            pl.BlockSpec((BLK, d_in), lambda t, sr, sc, se, sg: (sr[t], 0)),
            pl.BlockSpec((d_in, E_PAD), lambda t, sr, sc, se, sg: (0, 0)),
            pl.BlockSpec((1, E_PAD), lambda t, sr, sc, se, sg: (0, 0)),
            pl.BlockSpec((1, d_in, d_hid),
                         lambda t, sr, sc, se, sg: (sc[t], 0, 0)),
            pl.BlockSpec((1, 1, d_hid),
                         lambda t, sr, sc, se, sg: (sc[t], 0, 0)),
            pl.BlockSpec((1, d_hid, d_out),
                         lambda t, sr, sc, se, sg: (sc[t], 0, 0)),
            pl.BlockSpec((1, 1, d_out),
                         lambda t, sr, sc, se, sg: (sc[t], 0, 0)),
        ],
        out_specs=pl.BlockSpec((BLK, d_out),
                               lambda t, sr, sc, se, sg: (sr[t], 0)),
    )
    return pl.pallas_call(
        functools.partial(_mlp_body, n_experts),
        grid_spec=grid_spec,
        out_shape=jax.ShapeDtypeStruct((n_rows, d_out), jnp.float32),
        compiler_params=pltpu.CompilerParams(
            dimension_semantics=("arbitrary",)),
    )(tile_r, tile_ec, tile_e, gs8, xs, wg_pad, bg_pad, W1, b1r, W2, b2r)


def _combine(ysw, pos1, pos2, n_tok):
    n_rows, d_out = ysw.shape
    per_w = n_tok // NW
    n_ch = per_w // CH
    mesh = plsc.VectorSubcoreMesh(core_axis_name="c", subcore_axis_name="s")

    @functools.partial(
        pl.kernel, mesh=mesh,
        out_type=jax.ShapeDtypeStruct((n_tok, d_out), jnp.float32),
        scratch_types=[
            pltpu.VMEM((n_ch, CH), jnp.int32),
            pltpu.VMEM((n_ch, CH), jnp.int32),
            pltpu.VMEM((CH, d_out), jnp.float32),
            pltpu.VMEM((CH, d_out), jnp.float32),
            pltpu.VMEM((CH, d_out), jnp.float32),
            pltpu.VMEM((CH, d_out), jnp.float32),
            pltpu.SemaphoreType.DMA,
            pltpu.SemaphoreType.DMA,
            pltpu.SemaphoreType.DMA,
            pltpu.SemaphoreType.DMA,
        ],
    )
    def k4(ys_hbm, p1_hbm, p2_hbm, out_hbm, ia, ib, ba0, bb0, ba1, bb1,
           g0, g1, st0, st1):
        wid = lax.axis_index("s") * 2 + lax.axis_index("c")
        pltpu.sync_copy(p1_hbm.at[wid], ia)
        pltpu.sync_copy(p2_hbm.at[wid], ib)
        base = wid * per_w
        babufs = (ba0, ba1)
        bbbufs = (bb0, bb1)
        gsem = (g0, g1)
        stsem = (st0, st1)
        gaths = [None] * n_ch
        stores = [None] * n_ch

        def issue_gathers(c):
            p = c & 1
            h1 = pltpu.async_copy(ys_hbm.at[ia.at[c]], babufs[p], gsem[p])
            h2 = pltpu.async_copy(ys_hbm.at[ib.at[c]], bbbufs[p], gsem[p])
            gaths[c] = (h1, h2)

        issue_gathers(0)
        for c in range(n_ch):
            p = c & 1
            gaths[c][0].wait()
            gaths[c][1].wait()
            if c + 1 < n_ch:
                if c >= 1:
                    stores[c - 1].wait()
                issue_gathers(c + 1)
            ba = babufs[p]
            bb = bbbufs[p]

            def add_row(j, _):
                for k in range(d_out // 16):
                    sl = pl.ds(k * 16, 16)
                    ba[j, sl] = ba[j, sl] + bb[j, sl]
                return 0

            lax.fori_loop(0, CH, add_row, 0)
            stores[c] = pltpu.async_copy(
                ba, out_hbm.at[pl.ds(base + c * CH, CH)], stsem[p])
        if n_ch >= 2:
            stores[n_ch - 2].wait()
        stores[n_ch - 1].wait()

    return k4(ysw, pos1, pos2)


def kernel(x, Wg, bg, W1, b1, W2, b2):
    n_tok, d_in = x.shape
    n_experts, _, d_hid = W1.shape
    d_out = W2.shape[2]
    n_rows = 2 * n_tok
    n_rows_pad = n_rows + n_experts * BLK
    t_max = n_rows // BLK + n_experts - 1

    wg_pad = jnp.zeros((d_in, E_PAD), jnp.float32).at[:, :n_experts].set(Wg)
    bg_pad = jnp.full((E_PAD,), NEG, jnp.float32).at[:n_experts].set(bg)
    bg_pad = bg_pad.reshape(1, E_PAD)
    b1r = b1.reshape(n_experts, 1, d_hid)
    b2r = b2.reshape(n_experts, 1, d_out)

    p1f, p2f, counts_f = _gating(x, wg_pad, bg_pad)
    pos1 = p1f[:, 0].reshape(NW, -1)
    pos2 = p2f[:, 0].reshape(NW, -1)
    ch_d = 32 if (n_tok // NW) % 32 == 0 else CH
    pos1_d = pos1.reshape(NW, -1, ch_d)
    pos2_d = pos2.reshape(NW, -1, ch_d)
    pos1_c = pos1.reshape(NW, -1, CH)
    pos2_c = pos2.reshape(NW, -1, CH)

    counts = counts_f[0, :n_experts].astype(jnp.int32)
    nt = (counts + BLK - 1) // BLK
    pcnts = nt * BLK
    astarts = jnp.cumsum(pcnts) - pcnts
    gs8 = jnp.concatenate(
        [astarts, jnp.full((8 - n_experts,), n_rows_pad, jnp.int32)])
    first_t = astarts // BLK
    nt_ends = jnp.cumsum(nt)
    nt_starts = nt_ends - nt
    t_ids = jnp.arange(t_max, dtype=jnp.int32)
    e_t = jnp.sum(t_ids[:, None] >= nt_ends[None, :], axis=1).astype(jnp.int32)
    e_ids = jnp.arange(n_experts, dtype=jnp.int32)
    e_last = jnp.max(jnp.where(nt > 0, e_ids, 0))
    r_last = jnp.max(jnp.where(nt > 0, first_t + nt - 1, 0))
    e_m = jnp.minimum(e_t, n_experts - 1)
    e_c = jnp.where(e_t < n_experts, e_t, e_last).astype(jnp.int32)
    r_t = jnp.where(e_t < n_experts,
                    first_t[e_m] + (t_ids - nt_starts[e_m]),
                    r_last).astype(jnp.int32)

    xs = _dispatch(x, pos1_d, pos2_d, n_rows_pad)
    ysw = _grouped_mlp(xs, wg_pad, bg_pad, W1, b1r, W2, b2r,
                       r_t, e_c, e_t, gs8, t_max)
    return _combine(ysw, pos1_c, pos2_c, n_tok)
